# baseline (speedup 1.0000x reference)
import jax
import jax.numpy as jnp
from jax import lax
from jax.experimental import pallas as pl
from jax.experimental.pallas import tpu as pltpu
from jax.experimental.pallas import tpu_sc as plsc

N = 5000
NPAD = 5120
ROWS = 40
LANES = 128
SCORE_T = 0.3
NMS_T = 0.5

_NC = 2
_NS = 16
_NW = _NC * _NS
_PER_W = NPAD // _NW
_L = 16


def _sc_gather_body(boxes_hbm, scores_hbm, order_hbm,
                    xs_hbm, ys_hbm, ws_hbm, hs_hbm, ss_hbm,
                    x1_hbm, y1_hbm, ar_hbm,
                    btab, stab, idxv, xb, yb, wb, hb, sb, x1b, y1b, arb):
    wid = lax.axis_index("s") * _NC + lax.axis_index("c")
    base = wid * _PER_W
    pltpu.sync_copy(boxes_hbm, btab)
    pltpu.sync_copy(scores_hbm, stab)
    pltpu.sync_copy(order_hbm.at[pl.ds(base, _PER_W)], idxv)
    lanes = lax.iota(jnp.int32, _L)
    for v in range(_PER_W // _L):
        idx = idxv[pl.ds(v * _L, _L)]
        i4 = idx * 4
        x = plsc.load_gather(btab, [i4])
        y = plsc.load_gather(btab, [i4 + 1])
        w = plsc.load_gather(btab, [i4 + 2])
        h = plsc.load_gather(btab, [i4 + 3])
        s = plsc.load_gather(stab, [idx])
        pos = base + v * _L + lanes
        s = jnp.where(pos < N, s, jnp.float32(0.0))
        sl = pl.ds(v * _L, _L)
        xb[sl] = x
        yb[sl] = y
        wb[sl] = w
        hb[sl] = h
        sb[sl] = s
        x1b[sl] = x + w
        y1b[sl] = y + h
        arb[sl] = w * h
    pltpu.sync_copy(xb, xs_hbm.at[pl.ds(base, _PER_W)])
    pltpu.sync_copy(yb, ys_hbm.at[pl.ds(base, _PER_W)])
    pltpu.sync_copy(wb, ws_hbm.at[pl.ds(base, _PER_W)])
    pltpu.sync_copy(hb, hs_hbm.at[pl.ds(base, _PER_W)])
    pltpu.sync_copy(sb, ss_hbm.at[pl.ds(base, _PER_W)])
    pltpu.sync_copy(x1b, x1_hbm.at[pl.ds(base, _PER_W)])
    pltpu.sync_copy(y1b, y1_hbm.at[pl.ds(base, _PER_W)])
    pltpu.sync_copy(arb, ar_hbm.at[pl.ds(base, _PER_W)])


def _sc_gather(boxes_flat, scores, order_pad):
    vec = jax.ShapeDtypeStruct((NPAD,), jnp.float32)
    fbuf = pltpu.VMEM((_PER_W,), jnp.float32)
    run = pl.kernel(
        _sc_gather_body,
        out_type=(vec,) * 8,
        mesh=plsc.VectorSubcoreMesh(core_axis_name="c", subcore_axis_name="s"),
        compiler_params=pltpu.CompilerParams(needs_layout_passes=False),
        scratch_types=[
            pltpu.VMEM((N * 4,), jnp.float32),
            pltpu.VMEM((N,), jnp.float32),
            pltpu.VMEM((_PER_W,), jnp.int32),
            fbuf, fbuf, fbuf, fbuf, fbuf, fbuf, fbuf, fbuf,
        ],
    )
    return run(boxes_flat, scores, order_pad)


BLK = 1024
NBLK = NPAD // BLK
BROWS = BLK // LANES


def _tc_nms_body(xs_ref, ys_ref, ws_ref, hs_ref, ss_ref,
                 x1_ref, y1_ref, ar_ref,
                 xt_ref, yt_ref, x1t_ref, y1t_ref, art_ref,
                 out_ref, m_ref):
    s = ss_ref[...]
    validf = jnp.where(s > SCORE_T, jnp.float32(1.0), jnp.float32(0.0))
    keep_rows = []
    lane = lax.broadcasted_iota(jnp.int32, (1, LANES), 1).astype(jnp.float32)
    sub8 = lax.broadcasted_iota(jnp.int32, (BROWS, 1), 0).astype(jnp.float32)

    for b in range(NBLK):
        r0 = b * BROWS

        zeros8 = jnp.zeros((BROWS, LANES), jnp.float32)
        for rr8 in range(BROWS):
            rr = r0 + rr8
            ccs = list(range(rr, ROWS))
            groups = [ccs[i:i + 8] for i in range(0, len(ccs), 8)]
            for grp in groups:
                jvals = []
                for cc in grp:
                    jvals.append((
                        jnp.broadcast_to(xs_ref[cc:cc + 1, :], (BROWS, LANES)),
                        jnp.broadcast_to(ys_ref[cc:cc + 1, :], (BROWS, LANES)),
                        jnp.broadcast_to(x1_ref[cc:cc + 1, :], (BROWS, LANES)),
                        jnp.broadcast_to(y1_ref[cc:cc + 1, :], (BROWS, LANES)),
                        jnp.broadcast_to(ar_ref[cc:cc + 1, :], (BROWS, LANES)),
                    ))

                def gen(gi, _, rr=rr, rr8=rr8, grp=grp, jvals=jvals):
                    c0 = gi * BROWS
                    xi = jnp.broadcast_to(xt_ref[pl.ds(c0, 8), rr:rr + 1],
                                          (BROWS, LANES))
                    yi = jnp.broadcast_to(yt_ref[pl.ds(c0, 8), rr:rr + 1],
                                          (BROWS, LANES))
                    x1i = jnp.broadcast_to(x1t_ref[pl.ds(c0, 8), rr:rr + 1],
                                           (BROWS, LANES))
                    y1i = jnp.broadcast_to(y1t_ref[pl.ds(c0, 8), rr:rr + 1],
                                           (BROWS, LANES))
                    ari = jnp.broadcast_to(art_ref[pl.ds(c0, 8), rr:rr + 1],
                                           (BROWS, LANES))
                    row0 = pl.ds((rr8 * 16 + gi) * 8, 8)
                    for cc, (xj, yj, x1j, y1j, arj) in zip(grp, jvals):
                        ww = jnp.maximum(
                            jnp.minimum(x1i, x1j) - jnp.maximum(xi, xj), 0.0)
                        hh = jnp.maximum(
                            jnp.minimum(y1i, y1j) - jnp.maximum(yi, yj), 0.0)
                        inner = ww * hh
                        union = jnp.maximum(ari + arj - inner,
                                            jnp.float32(1e-6))
                        iou = inner / union
                        sup = iou > NMS_T
                        if cc == rr:
                            iflat = (jnp.float32(rr * LANES)
                                     + gi.astype(jnp.float32) * 8.0 + sub8)
                            jflat = jnp.float32(cc * LANES) + lane
                            sup = sup & (jflat > iflat)
                        m_ref[row0, pl.ds(cc * LANES, LANES)] = (
                            jnp.where(sup, jnp.float32(1.0), jnp.float32(0.0)))
                    return 0

                lax.fori_loop(0, 16, gen, 0, unroll=False)

            if rr8 > 0:
                def zgen(gi, _, rr8=rr8):
                    row0 = pl.ds((rr8 * 16 + gi) * 8, 8)
                    for cc in range(r0, r0 + rr8):
                        m_ref[row0, pl.ds(cc * LANES, LANES)] = zeros8
                    return 0

                lax.fori_loop(0, 16, zgen, 0, unroll=False)

        vblk = jnp.broadcast_to(
            validf[r0:r0 + BROWS, :].reshape(1, BLK), (BROWS, BLK))
        mblk = m_ref[:, pl.ds(b * BLK, BLK)]

        def sweep_cond(st):
            _, changed = st
            return changed > 0.0

        def sweep_body(st):
            k, _ = st
            sup = jnp.dot(k, mblk, preferred_element_type=jnp.float32)
            knew = jnp.where(sup >= 0.5, jnp.float32(0.0), vblk)
            changed = jnp.sum(jnp.abs(knew - k))
            return knew, changed

        kblk, _ = lax.while_loop(sweep_cond, sweep_body,
                                 (vblk, jnp.float32(1.0)))
        keep_rows.append(kblk[0:1, :].reshape(BROWS, LANES))

        if b < NBLK - 1:
            rest = NPAD - (b + 1) * BLK
            mrest = m_ref[:, pl.ds((b + 1) * BLK, rest)]
            supl = jnp.dot(kblk, mrest, preferred_element_type=jnp.float32)
            okl = jnp.where(supl[0:1, :] >= 0.5, jnp.float32(0.0),
                            jnp.float32(1.0)).reshape(ROWS - r0 - BROWS,
                                                      LANES)
            validf = jnp.concatenate(
                [validf[:r0 + BROWS, :],
                 validf[r0 + BROWS:, :] * okl], axis=0)

    kf = jnp.concatenate(keep_rows, axis=0)
    out_ref[0] = s * kf
    out_ref[1] = xs_ref[...] * kf
    out_ref[2] = ys_ref[...] * kf
    out_ref[3] = ws_ref[...] * kf
    out_ref[4] = hs_ref[...] * kf


def _tc_nms(xs, ys, ws, hs, ss, x1f, y1f, arf):
    x2 = xs.reshape(ROWS, LANES)
    y2 = ys.reshape(ROWS, LANES)
    x12 = x1f.reshape(ROWS, LANES)
    y12 = y1f.reshape(ROWS, LANES)
    ar2 = arf.reshape(ROWS, LANES)
    return pl.pallas_call(
        _tc_nms_body,
        out_shape=jax.ShapeDtypeStruct((5, ROWS, LANES), jnp.float32),
        scratch_shapes=[pltpu.VMEM((BLK, NPAD), jnp.float32)],
    )(x2, y2, ws.reshape(ROWS, LANES), hs.reshape(ROWS, LANES),
      ss.reshape(ROWS, LANES), x12, y12, ar2,
      x2.T, y2.T, x12.T, y12.T, ar2.T)


def kernel(boxes, scores):
    order = jnp.argsort(-scores).astype(jnp.int32)
    order_pad = jnp.concatenate(
        [order, jnp.zeros((NPAD - N,), jnp.int32)])
    xs, ys, ws, hs, ss, x1f, y1f, arf = _sc_gather(
        boxes.reshape(-1), scores, order_pad)
    out5 = _tc_nms(xs, ys, ws, hs, ss, x1f, y1f, arf)
    return out5.reshape(5, NPAD)[:, :N].T


# speedup vs baseline: 1.6774x; 1.6774x over previous
import jax
import jax.numpy as jnp
from jax import lax
from jax.experimental import pallas as pl
from jax.experimental.pallas import tpu as pltpu
from jax.experimental.pallas import tpu_sc as plsc

N = 5000
NPAD = 5120
ROWS = 40
LANES = 128
SCORE_T = 0.3
NMS_T = 0.5

_NC = 2
_NS = 16
_NW = _NC * _NS
_PER_W = NPAD // _NW
_L = 16


def _sc_gather_body(boxes_hbm, scores_hbm, order_hbm,
                    xs_hbm, ys_hbm, ws_hbm, hs_hbm, ss_hbm,
                    x1_hbm, y1_hbm, ar_hbm,
                    btab, stab, idxv, xb, yb, wb, hb, sb, x1b, y1b, arb):
    wid = lax.axis_index("s") * _NC + lax.axis_index("c")
    base = wid * _PER_W
    pltpu.sync_copy(boxes_hbm, btab)
    pltpu.sync_copy(scores_hbm, stab)
    pltpu.sync_copy(order_hbm.at[pl.ds(base, _PER_W)], idxv)
    lanes = lax.iota(jnp.int32, _L)
    for v in range(_PER_W // _L):
        idx = idxv[pl.ds(v * _L, _L)]
        i4 = idx * 4
        x = plsc.load_gather(btab, [i4])
        y = plsc.load_gather(btab, [i4 + 1])
        w = plsc.load_gather(btab, [i4 + 2])
        h = plsc.load_gather(btab, [i4 + 3])
        s = plsc.load_gather(stab, [idx])
        pos = base + v * _L + lanes
        s = jnp.where(pos < N, s, jnp.float32(0.0))
        sl = pl.ds(v * _L, _L)
        xb[sl] = x
        yb[sl] = y
        wb[sl] = w
        hb[sl] = h
        sb[sl] = s
        x1b[sl] = x + w
        y1b[sl] = y + h
        arb[sl] = w * h
    pltpu.sync_copy(xb, xs_hbm.at[pl.ds(base, _PER_W)])
    pltpu.sync_copy(yb, ys_hbm.at[pl.ds(base, _PER_W)])
    pltpu.sync_copy(wb, ws_hbm.at[pl.ds(base, _PER_W)])
    pltpu.sync_copy(hb, hs_hbm.at[pl.ds(base, _PER_W)])
    pltpu.sync_copy(sb, ss_hbm.at[pl.ds(base, _PER_W)])
    pltpu.sync_copy(x1b, x1_hbm.at[pl.ds(base, _PER_W)])
    pltpu.sync_copy(y1b, y1_hbm.at[pl.ds(base, _PER_W)])
    pltpu.sync_copy(arb, ar_hbm.at[pl.ds(base, _PER_W)])


def _sc_gather(boxes_flat, scores, order_pad):
    vec = jax.ShapeDtypeStruct((NPAD,), jnp.float32)
    fbuf = pltpu.VMEM((_PER_W,), jnp.float32)
    run = pl.kernel(
        _sc_gather_body,
        out_type=(vec,) * 8,
        mesh=plsc.VectorSubcoreMesh(core_axis_name="c", subcore_axis_name="s"),
        compiler_params=pltpu.CompilerParams(needs_layout_passes=False),
        scratch_types=[
            pltpu.VMEM((N * 4,), jnp.float32),
            pltpu.VMEM((N,), jnp.float32),
            pltpu.VMEM((_PER_W,), jnp.int32),
            fbuf, fbuf, fbuf, fbuf, fbuf, fbuf, fbuf, fbuf,
        ],
    )
    return run(boxes_flat, scores, order_pad)


BLK = 1024
NBLK = NPAD // BLK
BROWS = BLK // LANES


def _tc_nms_body(xs_ref, ys_ref, ws_ref, hs_ref, ss_ref,
                 x1_ref, y1_ref, ar_ref,
                 xt_ref, yt_ref, x1t_ref, y1t_ref, art_ref,
                 out_ref, m_ref):
    s = ss_ref[...]
    validf = jnp.where(s > SCORE_T, jnp.float32(1.0), jnp.float32(0.0))
    keep_rows = []
    lane = lax.broadcasted_iota(jnp.int32, (1, LANES), 1).astype(jnp.float32)
    sub8 = lax.broadcasted_iota(jnp.int32, (BROWS, 1), 0).astype(jnp.float32)

    for b in range(NBLK):
        r0 = b * BROWS

        zeros8 = jnp.zeros((BROWS, LANES), jnp.float32)
        for rr8 in range(BROWS):
            rr = r0 + rr8
            def gen(gi, _, rr=rr, rr8=rr8):
                c0 = gi * BROWS
                xi = jnp.broadcast_to(xt_ref[pl.ds(c0, 8), rr:rr + 1],
                                      (BROWS, LANES))
                yi = jnp.broadcast_to(yt_ref[pl.ds(c0, 8), rr:rr + 1],
                                      (BROWS, LANES))
                x1i = jnp.broadcast_to(x1t_ref[pl.ds(c0, 8), rr:rr + 1],
                                       (BROWS, LANES))
                y1i = jnp.broadcast_to(y1t_ref[pl.ds(c0, 8), rr:rr + 1],
                                       (BROWS, LANES))
                ari = jnp.broadcast_to(art_ref[pl.ds(c0, 8), rr:rr + 1],
                                       (BROWS, LANES))
                row0 = pl.ds((rr8 * 16 + gi) * 8, 8)
                for cc in range(rr, ROWS):
                    xj = jnp.broadcast_to(xs_ref[cc:cc + 1, :],
                                          (BROWS, LANES))
                    yj = jnp.broadcast_to(ys_ref[cc:cc + 1, :],
                                          (BROWS, LANES))
                    x1j = jnp.broadcast_to(x1_ref[cc:cc + 1, :],
                                           (BROWS, LANES))
                    y1j = jnp.broadcast_to(y1_ref[cc:cc + 1, :],
                                           (BROWS, LANES))
                    arj = jnp.broadcast_to(ar_ref[cc:cc + 1, :],
                                           (BROWS, LANES))
                    ww = jnp.maximum(
                        jnp.minimum(x1i, x1j) - jnp.maximum(xi, xj), 0.0)
                    hh = jnp.maximum(
                        jnp.minimum(y1i, y1j) - jnp.maximum(yi, yj), 0.0)
                    inner = ww * hh
                    union = jnp.maximum(ari + arj - inner, jnp.float32(1e-6))
                    iou = inner / union
                    sup = iou > NMS_T
                    if cc == rr:
                        iflat = (jnp.float32(rr * LANES)
                                 + gi.astype(jnp.float32) * 8.0 + sub8)
                        jflat = jnp.float32(cc * LANES) + lane
                        sup = sup & (jflat > iflat)
                    m_ref[row0, pl.ds(cc * LANES, LANES)] = (
                        jnp.where(sup, jnp.float32(1.0), jnp.float32(0.0)))
                return 0

            lax.fori_loop(0, 16, gen, 0, unroll=False)

            if rr8 > 0:
                def zgen(gi, _, rr8=rr8):
                    row0 = pl.ds((rr8 * 16 + gi) * 8, 8)
                    for cc in range(r0, r0 + rr8):
                        m_ref[row0, pl.ds(cc * LANES, LANES)] = zeros8
                    return 0

                lax.fori_loop(0, 16, zgen, 0, unroll=False)

        vblk = jnp.broadcast_to(
            validf[r0:r0 + BROWS, :].reshape(1, BLK), (BROWS, BLK))
        mblk = m_ref[:, pl.ds(b * BLK, BLK)]

        def sweep_cond(st):
            _, changed = st
            return changed > 0.0

        def sweep_body(st):
            k, _ = st
            sup = jnp.dot(k, mblk, preferred_element_type=jnp.float32)
            knew = jnp.where(sup >= 0.5, jnp.float32(0.0), vblk)
            changed = jnp.sum(jnp.abs(knew - k))
            return knew, changed

        kblk, _ = lax.while_loop(sweep_cond, sweep_body,
                                 (vblk, jnp.float32(1.0)))
        keep_rows.append(kblk[0:1, :].reshape(BROWS, LANES))

        if b < NBLK - 1:
            rest = NPAD - (b + 1) * BLK
            mrest = m_ref[:, pl.ds((b + 1) * BLK, rest)]
            supl = jnp.dot(kblk, mrest, preferred_element_type=jnp.float32)
            okl = jnp.where(supl[0:1, :] >= 0.5, jnp.float32(0.0),
                            jnp.float32(1.0)).reshape(ROWS - r0 - BROWS,
                                                      LANES)
            validf = jnp.concatenate(
                [validf[:r0 + BROWS, :],
                 validf[r0 + BROWS:, :] * okl], axis=0)

    kf = jnp.concatenate(keep_rows, axis=0)
    out_ref[0] = s * kf
    out_ref[1] = xs_ref[...] * kf
    out_ref[2] = ys_ref[...] * kf
    out_ref[3] = ws_ref[...] * kf
    out_ref[4] = hs_ref[...] * kf


def _tc_nms(xs, ys, ws, hs, ss, x1f, y1f, arf):
    x2 = xs.reshape(ROWS, LANES)
    y2 = ys.reshape(ROWS, LANES)
    x12 = x1f.reshape(ROWS, LANES)
    y12 = y1f.reshape(ROWS, LANES)
    ar2 = arf.reshape(ROWS, LANES)
    return pl.pallas_call(
        _tc_nms_body,
        out_shape=jax.ShapeDtypeStruct((5, ROWS, LANES), jnp.float32),
        scratch_shapes=[pltpu.VMEM((BLK, NPAD), jnp.float32)],
    )(x2, y2, ws.reshape(ROWS, LANES), hs.reshape(ROWS, LANES),
      ss.reshape(ROWS, LANES), x12, y12, ar2,
      x2.T, y2.T, x12.T, y12.T, ar2.T)


def kernel(boxes, scores):
    order = jnp.argsort(-scores).astype(jnp.int32)
    order_pad = jnp.concatenate(
        [order, jnp.zeros((NPAD - N,), jnp.int32)])
    xs, ys, ws, hs, ss, x1f, y1f, arf = _sc_gather(
        boxes.reshape(-1), scores, order_pad)
    out5 = _tc_nms(xs, ys, ws, hs, ss, x1f, y1f, arf)
    return out5.reshape(5, NPAD)[:, :N].T


# speedup vs baseline: 2.3418x; 1.3961x over previous
import jax
import jax.numpy as jnp
from jax import lax
from jax.experimental import pallas as pl
from jax.experimental.pallas import tpu as pltpu
from jax.experimental.pallas import tpu_sc as plsc

N = 5000
NPAD = 5120
ROWS = 40
LANES = 128
SCORE_T = 0.3
NMS_T = 0.5

_NC = 2
_NS = 16
_NW = _NC * _NS
_PER_W = NPAD // _NW
_L = 16


def _sc_gather_body(boxes_hbm, scores_hbm, order_hbm,
                    xs_hbm, ys_hbm, ws_hbm, hs_hbm, ss_hbm,
                    x1_hbm, y1_hbm, ar_hbm,
                    btab, stab, idxv, xb, yb, wb, hb, sb, x1b, y1b, arb):
    wid = lax.axis_index("s") * _NC + lax.axis_index("c")
    base = wid * _PER_W
    pltpu.sync_copy(boxes_hbm, btab)
    pltpu.sync_copy(scores_hbm, stab)
    pltpu.sync_copy(order_hbm.at[pl.ds(base, _PER_W)], idxv)
    lanes = lax.iota(jnp.int32, _L)
    for v in range(_PER_W // _L):
        idx = idxv[pl.ds(v * _L, _L)]
        i4 = idx * 4
        x = plsc.load_gather(btab, [i4])
        y = plsc.load_gather(btab, [i4 + 1])
        w = plsc.load_gather(btab, [i4 + 2])
        h = plsc.load_gather(btab, [i4 + 3])
        s = plsc.load_gather(stab, [idx])
        pos = base + v * _L + lanes
        s = jnp.where(pos < N, s, jnp.float32(0.0))
        sl = pl.ds(v * _L, _L)
        xb[sl] = x
        yb[sl] = y
        wb[sl] = w
        hb[sl] = h
        sb[sl] = s
        x1b[sl] = x + w
        y1b[sl] = y + h
        arb[sl] = w * h
    pltpu.sync_copy(xb, xs_hbm.at[pl.ds(base, _PER_W)])
    pltpu.sync_copy(yb, ys_hbm.at[pl.ds(base, _PER_W)])
    pltpu.sync_copy(wb, ws_hbm.at[pl.ds(base, _PER_W)])
    pltpu.sync_copy(hb, hs_hbm.at[pl.ds(base, _PER_W)])
    pltpu.sync_copy(sb, ss_hbm.at[pl.ds(base, _PER_W)])
    pltpu.sync_copy(x1b, x1_hbm.at[pl.ds(base, _PER_W)])
    pltpu.sync_copy(y1b, y1_hbm.at[pl.ds(base, _PER_W)])
    pltpu.sync_copy(arb, ar_hbm.at[pl.ds(base, _PER_W)])


def _sc_gather(boxes_flat, scores, order_pad):
    vec = jax.ShapeDtypeStruct((NPAD,), jnp.float32)
    fbuf = pltpu.VMEM((_PER_W,), jnp.float32)
    run = pl.kernel(
        _sc_gather_body,
        out_type=(vec,) * 8,
        mesh=plsc.VectorSubcoreMesh(core_axis_name="c", subcore_axis_name="s"),
        compiler_params=pltpu.CompilerParams(needs_layout_passes=False),
        scratch_types=[
            pltpu.VMEM((N * 4,), jnp.float32),
            pltpu.VMEM((N,), jnp.float32),
            pltpu.VMEM((_PER_W,), jnp.int32),
            fbuf, fbuf, fbuf, fbuf, fbuf, fbuf, fbuf, fbuf,
        ],
    )
    return run(boxes_flat, scores, order_pad)


BLK = 1024
NBLK = NPAD // BLK
BROWS = BLK // LANES


def _tc_nms_body(xs_ref, ys_ref, ws_ref, hs_ref, ss_ref,
                 x1_ref, y1_ref, ar_ref,
                 xt_ref, yt_ref, x1t_ref, y1t_ref, art_ref,
                 out_ref, m_ref):
    s = ss_ref[...]
    validf = jnp.where(s > SCORE_T, jnp.float32(1.0), jnp.float32(0.0))
    keep_rows = []
    lane = lax.broadcasted_iota(jnp.int32, (1, LANES), 1).astype(jnp.float32)
    sub8 = lax.broadcasted_iota(jnp.int32, (BROWS, 1), 0).astype(jnp.float32)

    for b in range(NBLK):
        r0 = b * BROWS

        zeros8 = jnp.zeros((BROWS, LANES), jnp.float32)
        for rr8 in range(BROWS):
            rr = r0 + rr8
            def gen(gi, _, rr=rr, rr8=rr8):
                c0 = gi * BROWS
                xi = jnp.broadcast_to(xt_ref[pl.ds(c0, 8), rr:rr + 1],
                                      (BROWS, LANES))
                yi = jnp.broadcast_to(yt_ref[pl.ds(c0, 8), rr:rr + 1],
                                      (BROWS, LANES))
                x1i = jnp.broadcast_to(x1t_ref[pl.ds(c0, 8), rr:rr + 1],
                                       (BROWS, LANES))
                y1i = jnp.broadcast_to(y1t_ref[pl.ds(c0, 8), rr:rr + 1],
                                       (BROWS, LANES))
                ari = jnp.broadcast_to(art_ref[pl.ds(c0, 8), rr:rr + 1],
                                       (BROWS, LANES))
                row0 = pl.ds((rr8 * 16 + gi) * 8, 8)
                for cc in range(rr, ROWS):
                    xj = jnp.broadcast_to(xs_ref[cc:cc + 1, :],
                                          (BROWS, LANES))
                    yj = jnp.broadcast_to(ys_ref[cc:cc + 1, :],
                                          (BROWS, LANES))
                    x1j = jnp.broadcast_to(x1_ref[cc:cc + 1, :],
                                           (BROWS, LANES))
                    y1j = jnp.broadcast_to(y1_ref[cc:cc + 1, :],
                                           (BROWS, LANES))
                    arj = jnp.broadcast_to(ar_ref[cc:cc + 1, :],
                                           (BROWS, LANES))
                    ww = jnp.maximum(
                        jnp.minimum(x1i, x1j) - jnp.maximum(xi, xj), 0.0)
                    hh = jnp.maximum(
                        jnp.minimum(y1i, y1j) - jnp.maximum(yi, yj), 0.0)
                    inner = ww * hh
                    union = jnp.maximum(ari + arj - inner, jnp.float32(1e-6))
                    iou = inner / union
                    sup = iou > NMS_T
                    if cc == rr:
                        iflat = (jnp.float32(rr * LANES)
                                 + gi.astype(jnp.float32) * 8.0 + sub8)
                        jflat = jnp.float32(cc * LANES) + lane
                        sup = sup & (jflat > iflat)
                    m_ref[row0, pl.ds(cc * LANES, LANES)] = (
                        jnp.where(sup, jnp.float32(1.0), jnp.float32(0.0)))
                return 0

            lax.fori_loop(0, 16, gen, 0, unroll=4)

            if rr8 > 0:
                def zgen(gi, _, rr8=rr8):
                    row0 = pl.ds((rr8 * 16 + gi) * 8, 8)
                    for cc in range(r0, r0 + rr8):
                        m_ref[row0, pl.ds(cc * LANES, LANES)] = zeros8
                    return 0

                lax.fori_loop(0, 16, zgen, 0, unroll=False)

        vblk = jnp.broadcast_to(
            validf[r0:r0 + BROWS, :].reshape(1, BLK), (BROWS, BLK))
        mblk = m_ref[:, pl.ds(b * BLK, BLK)]

        def sweep_cond(st):
            _, changed = st
            return changed > 0.0

        def sweep_body(st):
            k, _ = st
            sup = jnp.dot(k, mblk, preferred_element_type=jnp.float32)
            knew = jnp.where(sup >= 0.5, jnp.float32(0.0), vblk)
            changed = jnp.sum(jnp.abs(knew - k))
            return knew, changed

        kblk, _ = lax.while_loop(sweep_cond, sweep_body,
                                 (vblk, jnp.float32(1.0)))
        keep_rows.append(kblk[0:1, :].reshape(BROWS, LANES))

        if b < NBLK - 1:
            rest = NPAD - (b + 1) * BLK
            mrest = m_ref[:, pl.ds((b + 1) * BLK, rest)]
            supl = jnp.dot(kblk, mrest, preferred_element_type=jnp.float32)
            okl = jnp.where(supl[0:1, :] >= 0.5, jnp.float32(0.0),
                            jnp.float32(1.0)).reshape(ROWS - r0 - BROWS,
                                                      LANES)
            validf = jnp.concatenate(
                [validf[:r0 + BROWS, :],
                 validf[r0 + BROWS:, :] * okl], axis=0)

    kf = jnp.concatenate(keep_rows, axis=0)
    out_ref[0] = s * kf
    out_ref[1] = xs_ref[...] * kf
    out_ref[2] = ys_ref[...] * kf
    out_ref[3] = ws_ref[...] * kf
    out_ref[4] = hs_ref[...] * kf


def _tc_nms(xs, ys, ws, hs, ss, x1f, y1f, arf):
    x2 = xs.reshape(ROWS, LANES)
    y2 = ys.reshape(ROWS, LANES)
    x12 = x1f.reshape(ROWS, LANES)
    y12 = y1f.reshape(ROWS, LANES)
    ar2 = arf.reshape(ROWS, LANES)
    return pl.pallas_call(
        _tc_nms_body,
        out_shape=jax.ShapeDtypeStruct((5, ROWS, LANES), jnp.float32),
        scratch_shapes=[pltpu.VMEM((BLK, NPAD), jnp.float32)],
    )(x2, y2, ws.reshape(ROWS, LANES), hs.reshape(ROWS, LANES),
      ss.reshape(ROWS, LANES), x12, y12, ar2,
      x2.T, y2.T, x12.T, y12.T, ar2.T)


def kernel(boxes, scores):
    order = jnp.argsort(-scores).astype(jnp.int32)
    order_pad = jnp.concatenate(
        [order, jnp.zeros((NPAD - N,), jnp.int32)])
    xs, ys, ws, hs, ss, x1f, y1f, arf = _sc_gather(
        boxes.reshape(-1), scores, order_pad)
    out5 = _tc_nms(xs, ys, ws, hs, ss, x1f, y1f, arf)
    return out5.reshape(5, NPAD)[:, :N].T


# speedup vs baseline: 2.4485x; 1.0456x over previous
import jax
import jax.numpy as jnp
from jax import lax
from jax.experimental import pallas as pl
from jax.experimental.pallas import tpu as pltpu
from jax.experimental.pallas import tpu_sc as plsc

N = 5000
NPAD = 5120
ROWS = 40
LANES = 128
SCORE_T = 0.3
NMS_T = 0.5

_NC = 2
_NS = 16
_NW = _NC * _NS
_PER_W = NPAD // _NW
_L = 16


def _sc_gather_body(boxes_hbm, scores_hbm, order_hbm,
                    xs_hbm, ys_hbm, ws_hbm, hs_hbm, ss_hbm,
                    x1_hbm, y1_hbm, ar_hbm,
                    btab, stab, idxv, xb, yb, wb, hb, sb, x1b, y1b, arb):
    wid = lax.axis_index("s") * _NC + lax.axis_index("c")
    base = wid * _PER_W
    pltpu.sync_copy(boxes_hbm, btab)
    pltpu.sync_copy(scores_hbm, stab)
    pltpu.sync_copy(order_hbm.at[pl.ds(base, _PER_W)], idxv)
    lanes = lax.iota(jnp.int32, _L)
    for v in range(_PER_W // _L):
        idx = idxv[pl.ds(v * _L, _L)]
        i4 = idx * 4
        x = plsc.load_gather(btab, [i4])
        y = plsc.load_gather(btab, [i4 + 1])
        w = plsc.load_gather(btab, [i4 + 2])
        h = plsc.load_gather(btab, [i4 + 3])
        s = plsc.load_gather(stab, [idx])
        pos = base + v * _L + lanes
        s = jnp.where(pos < N, s, jnp.float32(0.0))
        sl = pl.ds(v * _L, _L)
        xb[sl] = x
        yb[sl] = y
        wb[sl] = w
        hb[sl] = h
        sb[sl] = s
        x1b[sl] = x + w
        y1b[sl] = y + h
        arb[sl] = w * h
    pltpu.sync_copy(xb, xs_hbm.at[pl.ds(base, _PER_W)])
    pltpu.sync_copy(yb, ys_hbm.at[pl.ds(base, _PER_W)])
    pltpu.sync_copy(wb, ws_hbm.at[pl.ds(base, _PER_W)])
    pltpu.sync_copy(hb, hs_hbm.at[pl.ds(base, _PER_W)])
    pltpu.sync_copy(sb, ss_hbm.at[pl.ds(base, _PER_W)])
    pltpu.sync_copy(x1b, x1_hbm.at[pl.ds(base, _PER_W)])
    pltpu.sync_copy(y1b, y1_hbm.at[pl.ds(base, _PER_W)])
    pltpu.sync_copy(arb, ar_hbm.at[pl.ds(base, _PER_W)])


def _sc_gather(boxes_flat, scores, order_pad):
    vec = jax.ShapeDtypeStruct((NPAD,), jnp.float32)
    fbuf = pltpu.VMEM((_PER_W,), jnp.float32)
    run = pl.kernel(
        _sc_gather_body,
        out_type=(vec,) * 8,
        mesh=plsc.VectorSubcoreMesh(core_axis_name="c", subcore_axis_name="s"),
        compiler_params=pltpu.CompilerParams(needs_layout_passes=False),
        scratch_types=[
            pltpu.VMEM((N * 4,), jnp.float32),
            pltpu.VMEM((N,), jnp.float32),
            pltpu.VMEM((_PER_W,), jnp.int32),
            fbuf, fbuf, fbuf, fbuf, fbuf, fbuf, fbuf, fbuf,
        ],
    )
    return run(boxes_flat, scores, order_pad)


BLK = 1024
NBLK = NPAD // BLK
BROWS = BLK // LANES


def _tc_nms_body(xs_ref, ys_ref, ws_ref, hs_ref, ss_ref,
                 x1_ref, y1_ref, ar_ref,
                 xt_ref, yt_ref, x1t_ref, y1t_ref, art_ref,
                 out_ref, m_ref):
    s = ss_ref[...]
    validf = jnp.where(s > SCORE_T, jnp.float32(1.0), jnp.float32(0.0))
    keep_rows = []
    lane = lax.broadcasted_iota(jnp.int32, (1, LANES), 1).astype(jnp.float32)
    sub8 = lax.broadcasted_iota(jnp.int32, (BROWS, 1), 0).astype(jnp.float32)

    for b in range(NBLK):
        r0 = b * BROWS

        zeros8 = jnp.zeros((BROWS, LANES), jnp.float32)
        for rr8 in range(BROWS):
            rr = r0 + rr8
            def gen(gi, _, rr=rr, rr8=rr8):
                c0 = gi * BROWS
                xi = jnp.broadcast_to(xt_ref[pl.ds(c0, 8), rr:rr + 1],
                                      (BROWS, LANES))
                yi = jnp.broadcast_to(yt_ref[pl.ds(c0, 8), rr:rr + 1],
                                      (BROWS, LANES))
                x1i = jnp.broadcast_to(x1t_ref[pl.ds(c0, 8), rr:rr + 1],
                                       (BROWS, LANES))
                y1i = jnp.broadcast_to(y1t_ref[pl.ds(c0, 8), rr:rr + 1],
                                       (BROWS, LANES))
                ari = jnp.broadcast_to(art_ref[pl.ds(c0, 8), rr:rr + 1],
                                       (BROWS, LANES))
                row0 = pl.ds((rr8 * 16 + gi) * 8, 8)
                for cc in range(rr, ROWS):
                    xj = jnp.broadcast_to(xs_ref[cc:cc + 1, :],
                                          (BROWS, LANES))
                    yj = jnp.broadcast_to(ys_ref[cc:cc + 1, :],
                                          (BROWS, LANES))
                    x1j = jnp.broadcast_to(x1_ref[cc:cc + 1, :],
                                           (BROWS, LANES))
                    y1j = jnp.broadcast_to(y1_ref[cc:cc + 1, :],
                                           (BROWS, LANES))
                    arj = jnp.broadcast_to(ar_ref[cc:cc + 1, :],
                                           (BROWS, LANES))
                    ww = jnp.maximum(
                        jnp.minimum(x1i, x1j) - jnp.maximum(xi, xj), 0.0)
                    hh = jnp.maximum(
                        jnp.minimum(y1i, y1j) - jnp.maximum(yi, yj), 0.0)
                    inner = ww * hh
                    union = jnp.maximum(ari + arj - inner, jnp.float32(1e-6))
                    iou = inner / union
                    sup = iou > NMS_T
                    if cc == rr:
                        iflat = (jnp.float32(rr * LANES)
                                 + gi.astype(jnp.float32) * 8.0 + sub8)
                        jflat = jnp.float32(cc * LANES) + lane
                        sup = sup & (jflat > iflat)
                    m_ref[row0, pl.ds(cc * LANES, LANES)] = (
                        jnp.where(sup, jnp.float32(1.0), jnp.float32(0.0)))
                return 0

            lax.fori_loop(0, 16, gen, 0, unroll=8)

            if rr8 > 0:
                def zgen(gi, _, rr8=rr8):
                    row0 = pl.ds((rr8 * 16 + gi) * 8, 8)
                    for cc in range(r0, r0 + rr8):
                        m_ref[row0, pl.ds(cc * LANES, LANES)] = zeros8
                    return 0

                lax.fori_loop(0, 16, zgen, 0, unroll=False)

        vblk = jnp.broadcast_to(
            validf[r0:r0 + BROWS, :].reshape(1, BLK), (BROWS, BLK))
        mblk = m_ref[:, pl.ds(b * BLK, BLK)]

        def sweep_cond(st):
            _, changed = st
            return changed > 0.0

        def sweep_body(st):
            k, _ = st
            sup = jnp.dot(k, mblk, preferred_element_type=jnp.float32)
            knew = jnp.where(sup >= 0.5, jnp.float32(0.0), vblk)
            changed = jnp.sum(jnp.abs(knew - k))
            return knew, changed

        kblk, _ = lax.while_loop(sweep_cond, sweep_body,
                                 (vblk, jnp.float32(1.0)))
        keep_rows.append(kblk[0:1, :].reshape(BROWS, LANES))

        if b < NBLK - 1:
            rest = NPAD - (b + 1) * BLK
            mrest = m_ref[:, pl.ds((b + 1) * BLK, rest)]
            supl = jnp.dot(kblk, mrest, preferred_element_type=jnp.float32)
            okl = jnp.where(supl[0:1, :] >= 0.5, jnp.float32(0.0),
                            jnp.float32(1.0)).reshape(ROWS - r0 - BROWS,
                                                      LANES)
            validf = jnp.concatenate(
                [validf[:r0 + BROWS, :],
                 validf[r0 + BROWS:, :] * okl], axis=0)

    kf = jnp.concatenate(keep_rows, axis=0)
    out_ref[0] = s * kf
    out_ref[1] = xs_ref[...] * kf
    out_ref[2] = ys_ref[...] * kf
    out_ref[3] = ws_ref[...] * kf
    out_ref[4] = hs_ref[...] * kf


def _tc_nms(xs, ys, ws, hs, ss, x1f, y1f, arf):
    x2 = xs.reshape(ROWS, LANES)
    y2 = ys.reshape(ROWS, LANES)
    x12 = x1f.reshape(ROWS, LANES)
    y12 = y1f.reshape(ROWS, LANES)
    ar2 = arf.reshape(ROWS, LANES)
    return pl.pallas_call(
        _tc_nms_body,
        out_shape=jax.ShapeDtypeStruct((5, ROWS, LANES), jnp.float32),
        scratch_shapes=[pltpu.VMEM((BLK, NPAD), jnp.float32)],
    )(x2, y2, ws.reshape(ROWS, LANES), hs.reshape(ROWS, LANES),
      ss.reshape(ROWS, LANES), x12, y12, ar2,
      x2.T, y2.T, x12.T, y12.T, ar2.T)


def kernel(boxes, scores):
    order = jnp.argsort(-scores).astype(jnp.int32)
    order_pad = jnp.concatenate(
        [order, jnp.zeros((NPAD - N,), jnp.int32)])
    xs, ys, ws, hs, ss, x1f, y1f, arf = _sc_gather(
        boxes.reshape(-1), scores, order_pad)
    out5 = _tc_nms(xs, ys, ws, hs, ss, x1f, y1f, arf)
    return out5.reshape(5, NPAD)[:, :N].T


# speedup vs baseline: 2.5654x; 1.0477x over previous
import jax
import jax.numpy as jnp
from jax import lax
from jax.experimental import pallas as pl
from jax.experimental.pallas import tpu as pltpu
from jax.experimental.pallas import tpu_sc as plsc

N = 5000
NPAD = 5120
ROWS = 40
LANES = 128
SCORE_T = 0.3
NMS_T = 0.5

_NC = 2
_NS = 16
_NW = _NC * _NS
_PER_W = NPAD // _NW
_L = 16


def _sc_gather_body(boxes_hbm, scores_hbm, order_hbm,
                    xs_hbm, ys_hbm, ws_hbm, hs_hbm, ss_hbm,
                    x1_hbm, y1_hbm, ar_hbm,
                    btab, stab, idxv, xb, yb, wb, hb, sb, x1b, y1b, arb):
    wid = lax.axis_index("s") * _NC + lax.axis_index("c")
    base = wid * _PER_W
    pltpu.sync_copy(boxes_hbm, btab)
    pltpu.sync_copy(scores_hbm, stab)
    pltpu.sync_copy(order_hbm.at[pl.ds(base, _PER_W)], idxv)
    lanes = lax.iota(jnp.int32, _L)
    for v in range(_PER_W // _L):
        idx = idxv[pl.ds(v * _L, _L)]
        i4 = idx * 4
        x = plsc.load_gather(btab, [i4])
        y = plsc.load_gather(btab, [i4 + 1])
        w = plsc.load_gather(btab, [i4 + 2])
        h = plsc.load_gather(btab, [i4 + 3])
        s = plsc.load_gather(stab, [idx])
        pos = base + v * _L + lanes
        s = jnp.where(pos < N, s, jnp.float32(0.0))
        sl = pl.ds(v * _L, _L)
        xb[sl] = x
        yb[sl] = y
        wb[sl] = w
        hb[sl] = h
        sb[sl] = s
        x1b[sl] = x + w
        y1b[sl] = y + h
        arb[sl] = w * h
    pltpu.sync_copy(xb, xs_hbm.at[pl.ds(base, _PER_W)])
    pltpu.sync_copy(yb, ys_hbm.at[pl.ds(base, _PER_W)])
    pltpu.sync_copy(wb, ws_hbm.at[pl.ds(base, _PER_W)])
    pltpu.sync_copy(hb, hs_hbm.at[pl.ds(base, _PER_W)])
    pltpu.sync_copy(sb, ss_hbm.at[pl.ds(base, _PER_W)])
    pltpu.sync_copy(x1b, x1_hbm.at[pl.ds(base, _PER_W)])
    pltpu.sync_copy(y1b, y1_hbm.at[pl.ds(base, _PER_W)])
    pltpu.sync_copy(arb, ar_hbm.at[pl.ds(base, _PER_W)])


def _sc_gather(boxes_flat, scores, order_pad):
    vec = jax.ShapeDtypeStruct((NPAD,), jnp.float32)
    fbuf = pltpu.VMEM((_PER_W,), jnp.float32)
    run = pl.kernel(
        _sc_gather_body,
        out_type=(vec,) * 8,
        mesh=plsc.VectorSubcoreMesh(core_axis_name="c", subcore_axis_name="s"),
        compiler_params=pltpu.CompilerParams(needs_layout_passes=False),
        scratch_types=[
            pltpu.VMEM((N * 4,), jnp.float32),
            pltpu.VMEM((N,), jnp.float32),
            pltpu.VMEM((_PER_W,), jnp.int32),
            fbuf, fbuf, fbuf, fbuf, fbuf, fbuf, fbuf, fbuf,
        ],
    )
    return run(boxes_flat, scores, order_pad)


BLK = 1024
NBLK = NPAD // BLK
BROWS = BLK // LANES


def _tc_nms_body(xs_ref, ys_ref, ws_ref, hs_ref, ss_ref,
                 x1_ref, y1_ref, ar_ref,
                 xt_ref, yt_ref, x1t_ref, y1t_ref, art_ref,
                 out_ref, m_ref):
    s = ss_ref[...]
    validf = jnp.where(s > SCORE_T, jnp.float32(1.0), jnp.float32(0.0))
    keep_rows = []
    lane = lax.broadcasted_iota(jnp.int32, (1, LANES), 1).astype(jnp.float32)
    sub8 = lax.broadcasted_iota(jnp.int32, (BROWS, 1), 0).astype(jnp.float32)

    for b in range(NBLK):
        r0 = b * BROWS

        zeros8 = jnp.zeros((BROWS, LANES), jnp.float32)
        for rr8 in range(BROWS):
            rr = r0 + rr8
            def gen(gi, _, rr=rr, rr8=rr8):
                c0 = gi * BROWS
                xi = jnp.broadcast_to(xt_ref[pl.ds(c0, 8), rr:rr + 1],
                                      (BROWS, LANES))
                yi = jnp.broadcast_to(yt_ref[pl.ds(c0, 8), rr:rr + 1],
                                      (BROWS, LANES))
                x1i = jnp.broadcast_to(x1t_ref[pl.ds(c0, 8), rr:rr + 1],
                                       (BROWS, LANES))
                y1i = jnp.broadcast_to(y1t_ref[pl.ds(c0, 8), rr:rr + 1],
                                       (BROWS, LANES))
                ari = jnp.broadcast_to(art_ref[pl.ds(c0, 8), rr:rr + 1],
                                       (BROWS, LANES))
                row0 = pl.ds((rr8 * 16 + gi) * 8, 8)
                for cc in range(rr, ROWS):
                    xj = jnp.broadcast_to(xs_ref[cc:cc + 1, :],
                                          (BROWS, LANES))
                    yj = jnp.broadcast_to(ys_ref[cc:cc + 1, :],
                                          (BROWS, LANES))
                    x1j = jnp.broadcast_to(x1_ref[cc:cc + 1, :],
                                           (BROWS, LANES))
                    y1j = jnp.broadcast_to(y1_ref[cc:cc + 1, :],
                                           (BROWS, LANES))
                    arj = jnp.broadcast_to(ar_ref[cc:cc + 1, :],
                                           (BROWS, LANES))
                    ww = jnp.maximum(
                        jnp.minimum(x1i, x1j) - jnp.maximum(xi, xj), 0.0)
                    hh = jnp.maximum(
                        jnp.minimum(y1i, y1j) - jnp.maximum(yi, yj), 0.0)
                    inner = ww * hh
                    union = jnp.maximum(ari + arj - inner, jnp.float32(1e-6))
                    iou = inner / union
                    sup = iou > NMS_T
                    if cc == rr:
                        iflat = (jnp.float32(rr * LANES)
                                 + gi.astype(jnp.float32) * 8.0 + sub8)
                        jflat = jnp.float32(cc * LANES) + lane
                        sup = sup & (jflat > iflat)
                    m_ref[row0, pl.ds(cc * LANES, LANES)] = (
                        jnp.where(sup, jnp.float32(1.0), jnp.float32(0.0)))
                return 0

            lax.fori_loop(0, 16, gen, 0, unroll=16)

            if rr8 > 0:
                def zgen(gi, _, rr8=rr8):
                    row0 = pl.ds((rr8 * 16 + gi) * 8, 8)
                    for cc in range(r0, r0 + rr8):
                        m_ref[row0, pl.ds(cc * LANES, LANES)] = zeros8
                    return 0

                lax.fori_loop(0, 16, zgen, 0, unroll=False)

        vblk = jnp.broadcast_to(
            validf[r0:r0 + BROWS, :].reshape(1, BLK), (BROWS, BLK))
        mblk = m_ref[:, pl.ds(b * BLK, BLK)]

        def sweep_cond(st):
            _, changed = st
            return changed > 0.0

        def sweep_body(st):
            k, _ = st
            sup = jnp.dot(k, mblk, preferred_element_type=jnp.float32)
            knew = jnp.where(sup >= 0.5, jnp.float32(0.0), vblk)
            changed = jnp.sum(jnp.abs(knew - k))
            return knew, changed

        kblk, _ = lax.while_loop(sweep_cond, sweep_body,
                                 (vblk, jnp.float32(1.0)))
        keep_rows.append(kblk[0:1, :].reshape(BROWS, LANES))

        if b < NBLK - 1:
            rest = NPAD - (b + 1) * BLK
            mrest = m_ref[:, pl.ds((b + 1) * BLK, rest)]
            supl = jnp.dot(kblk, mrest, preferred_element_type=jnp.float32)
            okl = jnp.where(supl[0:1, :] >= 0.5, jnp.float32(0.0),
                            jnp.float32(1.0)).reshape(ROWS - r0 - BROWS,
                                                      LANES)
            validf = jnp.concatenate(
                [validf[:r0 + BROWS, :],
                 validf[r0 + BROWS:, :] * okl], axis=0)

    kf = jnp.concatenate(keep_rows, axis=0)
    out_ref[0] = s * kf
    out_ref[1] = xs_ref[...] * kf
    out_ref[2] = ys_ref[...] * kf
    out_ref[3] = ws_ref[...] * kf
    out_ref[4] = hs_ref[...] * kf


def _tc_nms(xs, ys, ws, hs, ss, x1f, y1f, arf):
    x2 = xs.reshape(ROWS, LANES)
    y2 = ys.reshape(ROWS, LANES)
    x12 = x1f.reshape(ROWS, LANES)
    y12 = y1f.reshape(ROWS, LANES)
    ar2 = arf.reshape(ROWS, LANES)
    return pl.pallas_call(
        _tc_nms_body,
        out_shape=jax.ShapeDtypeStruct((5, ROWS, LANES), jnp.float32),
        scratch_shapes=[pltpu.VMEM((BLK, NPAD), jnp.float32)],
    )(x2, y2, ws.reshape(ROWS, LANES), hs.reshape(ROWS, LANES),
      ss.reshape(ROWS, LANES), x12, y12, ar2,
      x2.T, y2.T, x12.T, y12.T, ar2.T)


def kernel(boxes, scores):
    order = jnp.argsort(-scores).astype(jnp.int32)
    order_pad = jnp.concatenate(
        [order, jnp.zeros((NPAD - N,), jnp.int32)])
    xs, ys, ws, hs, ss, x1f, y1f, arf = _sc_gather(
        boxes.reshape(-1), scores, order_pad)
    out5 = _tc_nms(xs, ys, ws, hs, ss, x1f, y1f, arf)
    return out5.reshape(5, NPAD)[:, :N].T
